# baseline (speedup 1.0000x reference)
import functools

import jax
import jax.numpy as jnp
from jax import lax
from jax.experimental import pallas as pl
from jax.experimental.pallas import tpu as pltpu
from jax.experimental.pallas import tpu_sc as plsc

N = 10000
E = 320000
D = 128
H = 128
L = 20
B = 1024
NCLS = 10

SC_CORES = 2
SC_SUB = 16
NW = SC_CORES * SC_SUB
K = 128
EPT = E // NW
NCH = 84
EPT_P = NCH * K
RPT = 624
RTAIL = N - RPT * SC_SUB
GB = B * L // SC_SUB
GCH = GB // K

f32 = jnp.float32


def _tc_in_body(x_ref, wcat_ref, bcat_ref, n_ref, a1_ref, t1_ref):
    y = jnp.dot(x_ref[...], wcat_ref[...], preferred_element_type=f32)
    y = y + bcat_ref[...]
    a1_ref[...] = y[:, :H] * n_ref[...]
    t1_ref[...] = y[:, H:]


def _tc_in(A1, wcat, bcat, n):
    blk = 1000
    return pl.pallas_call(
        _tc_in_body,
        grid=(N // blk,),
        in_specs=[
            pl.BlockSpec((blk, D), lambda i: (i, 0)),
            pl.BlockSpec((D, 2 * H), lambda i: (0, 0)),
            pl.BlockSpec((1, 2 * H), lambda i: (0, 0)),
            pl.BlockSpec((blk, 1), lambda i: (i, 0)),
        ],
        out_specs=[
            pl.BlockSpec((blk, H), lambda i: (i, 0)),
            pl.BlockSpec((blk, H), lambda i: (i, 0)),
        ],
        out_shape=[
            jax.ShapeDtypeStruct((N, H), f32),
            jax.ShapeDtypeStruct((N, H), f32),
        ],
    )(A1, wcat, bcat, n)


def _tc_mid_body(p0_ref, p1_ref, a1_ref, n_ref, w_ref, b_ref, t2_ref):
    nn = n_ref[...]
    x1 = a1_ref[...] + (p0_ref[0] + p1_ref[0]) * (1.0 - nn)
    t2_ref[...] = jnp.dot(x1, w_ref[...], preferred_element_type=f32) + b_ref[...]


def _tc_mid(p, a1, n, w, b):
    blk = 1000
    return pl.pallas_call(
        _tc_mid_body,
        grid=(N // blk,),
        in_specs=[
            pl.BlockSpec((1, blk, H), lambda i: (0, i, 0)),
            pl.BlockSpec((1, blk, H), lambda i: (1, i, 0)),
            pl.BlockSpec((blk, H), lambda i: (i, 0)),
            pl.BlockSpec((blk, 1), lambda i: (i, 0)),
            pl.BlockSpec((H, H), lambda i: (0, 0)),
            pl.BlockSpec((1, H), lambda i: (0, 0)),
        ],
        out_specs=pl.BlockSpec((blk, H), lambda i: (i, 0)),
        out_shape=jax.ShapeDtypeStruct((N, H), f32),
    )(p, p, a1, n, w, b)


def _tc_head_body(s_ref, w2_ref, b2_ref, w1r_ref, l1b_ref, cls_ref, clsb_ref,
                  z_ref, am_ref):
    w2 = w2_ref[...]
    b2 = b2_ref[...]
    z = jnp.broadcast_to(l1b_ref[...], (B, H))
    ssum = jnp.zeros((B, H), f32)
    for j in range(L):
        s = s_ref[0, j] + s_ref[1, j]
        ssum = ssum + s
        y = jnp.dot(s, w2, preferred_element_type=f32) + b2
        z = z + jnp.dot(y, w1r_ref[j], preferred_element_type=f32)
    y = jnp.dot(ssum, w2, preferred_element_type=f32) + b2
    z = z + jnp.dot(y, w1r_ref[L], preferred_element_type=f32)
    z_ref[...] = z
    logits = jnp.dot(z, cls_ref[...], preferred_element_type=f32) + clsb_ref[...]
    am_ref[...] = jnp.argmax(logits, axis=-1).astype(jnp.int32)[None, :]


def _tc_head(selp, w2, b2, w1r, l1b, cls, clsb):
    return pl.pallas_call(
        _tc_head_body,
        out_shape=[
            jax.ShapeDtypeStruct((B, H), f32),
            jax.ShapeDtypeStruct((1, B), jnp.int32),
        ],
    )(selp, w2, b2, w1r, l1b, cls, clsb)


def _sc_mesh():
    return plsc.VectorSubcoreMesh(core_axis_name="c", subcore_axis_name="s",
                                  num_cores=SC_CORES, num_subcores=SC_SUB)


def _zero_acc(zbuf, acc, sid):
    def zrow(i, carry):
        for hh in range(H // 16):
            zbuf[i, pl.ds(hh * 16, 16)] = jnp.zeros((16,), f32)
        return carry

    lax.fori_loop(0, K, zrow, 0)
    r0 = sid * RPT
    nfull = RPT // K
    rem = RPT % K
    for kb in range(nfull):
        pltpu.sync_copy(zbuf, acc.at[pl.ds(r0 + kb * K, K)])
    if rem:
        pltpu.sync_copy(zbuf.at[pl.ds(0, rem)],
                        acc.at[pl.ds(r0 + nfull * K, rem)])

    @pl.when(sid == SC_SUB - 1)
    def _():
        pltpu.sync_copy(zbuf.at[pl.ds(0, RTAIL)],
                        acc.at[pl.ds(RPT * SC_SUB, RTAIL)])


def _scale_chunk(rb, wf, c3):
    woff = c3 * K

    def grp(g, carry):
        e0 = g * 8
        for j in range(8):
            wv = plsc.load_gather(
                wf, [jnp.full((16,), woff + e0 + j, jnp.int32)])
            for hh in range(H // 16):
                sl = pl.ds(hh * 16, 16)
                rb[e0 + j, sl] = rb[e0 + j, sl] * wv
        return carry

    lax.fori_loop(0, K // 8, grp, 0, unroll=2)


def _spmm_accumulate(t_hbm, colrow_hbm, w_hbm, wid, cf, rw, wf, rbufs, acc,
                     psems, gsems, ssems):
    def col_prefetch(c, s6):
        pltpu.async_copy(colrow_hbm.at[wid, c, 0],
                         cf.at[pl.ds(s6 * K, K)], psems[s6])

    def wait_col(s6):
        pltpu.make_async_copy(colrow_hbm.at[wid, 0, 0],
                              cf.at[pl.ds(0, K)], psems[s6]).wait()

    def bundle(c, b3):
        idx = cf.at[pl.ds((c % 6) * K, K)]
        pltpu.async_copy(t_hbm.at[idx], rbufs[b3], gsems[b3])
        pltpu.async_copy(colrow_hbm.at[wid, c, 1], rw.at[b3], gsems[b3])
        pltpu.async_copy(w_hbm.at[wid, c], wf.at[pl.ds(b3 * K, K)], gsems[b3])

    def wait_bundle(b3):
        pltpu.make_async_copy(t_hbm.at[pl.ds(0, K)], rbufs[b3],
                              gsems[b3]).wait()
        pltpu.make_async_copy(colrow_hbm.at[wid, 0, 1], rw.at[b3],
                              gsems[b3]).wait()
        pltpu.make_async_copy(w_hbm.at[wid, 0], wf.at[pl.ds(0, K)],
                              gsems[b3]).wait()

    def scatter(c, b3):
        pltpu.sync_copy(rbufs[b3], acc.at[rw.at[b3]], add=True)

    def wait_scatter(b3):
        pltpu.make_async_copy(t_hbm.at[pl.ds(0, K)], rbufs[b3],
                              ssems[b3]).wait()

    for c in range(4):
        col_prefetch(c, c)
    for c in range(2):
        wait_col(c)
        bundle(c, c)

    def step(k, carry):
        for bb in range(6):
            c = 6 * k + bb
            b3 = bb % 3
            b2 = (bb + 2) % 3
            wait_bundle(b3)
            _scale_chunk(rbufs[b3], wf, b3)

            @pl.when(c + 2 < NCH)
            def _():
                wait_col((bb + 2) % 6)
                bundle(c + 2, b2)

            scatter(c, b3)

            @pl.when(c + 4 < NCH)
            def _():
                col_prefetch(c + 4, (bb + 4) % 6)
        return carry

    lax.fori_loop(0, NCH // 6, step, 0)


def _acc_writeout(acc, out_hbm, cid, sid):
    r0 = sid * RPT
    nfull = RPT // K
    rem = RPT % K
    for kb in range(nfull):
        sl = pl.ds(r0 + kb * K, K)
        pltpu.sync_copy(acc.at[sl], out_hbm.at[cid, sl])
    if rem:
        sl = pl.ds(r0 + nfull * K, rem)
        pltpu.sync_copy(acc.at[sl], out_hbm.at[cid, sl])

    @pl.when(sid == SC_SUB - 1)
    def _():
        sl = pl.ds(RPT * SC_SUB, RTAIL)
        pltpu.sync_copy(acc.at[sl], out_hbm.at[cid, sl])


_SC_SCRATCH = [
    pltpu.VMEM((6 * K,), jnp.int32),
    pltpu.VMEM((3, K), jnp.int32),
    pltpu.VMEM((3 * K,), f32),
    pltpu.VMEM((K, H), f32),
    pltpu.VMEM((K, H), f32),
    pltpu.VMEM((K, H), f32),
    pltpu.VMEM_SHARED((N, H), f32),
    pltpu.SemaphoreType.DMA,
    pltpu.SemaphoreType.DMA,
    pltpu.SemaphoreType.DMA,
    pltpu.SemaphoreType.DMA,
    pltpu.SemaphoreType.DMA,
    pltpu.SemaphoreType.DMA,
    pltpu.SemaphoreType.DMA,
    pltpu.SemaphoreType.DMA,
    pltpu.SemaphoreType.DMA,
    pltpu.SemaphoreType.DMA,
    pltpu.SemaphoreType.DMA,
    pltpu.SemaphoreType.DMA,
]


def _make_spmm1():
    @functools.partial(
        pl.kernel,
        mesh=_sc_mesh(),
        compiler_params=pltpu.CompilerParams(needs_layout_passes=False),
        out_type=jax.ShapeDtypeStruct((SC_CORES, N, H), f32),
        scratch_types=list(_SC_SCRATCH),
    )
    def spmm1(t_hbm, colrow_hbm, w_hbm, out_hbm,
              cf, rw, wf, rb0, rb1, rb2, acc,
              ps0, ps1, ps2, ps3, ps4, ps5,
              gs0, gs1, gs2, ss0, ss1, ss2):
        cid = lax.axis_index("c")
        sid = lax.axis_index("s")
        wid = cid * SC_SUB + sid

        _zero_acc(rb0, acc, sid)
        plsc.subcore_barrier()

        _spmm_accumulate(t_hbm, colrow_hbm, w_hbm, wid, cf, rw, wf,
                         [rb0, rb1, rb2], acc,
                         [ps0, ps1, ps2, ps3, ps4, ps5],
                         [gs0, gs1, gs2], [ss0, ss1, ss2])
        plsc.subcore_barrier()
        _acc_writeout(acc, out_hbm, cid, sid)

    return spmm1


def _make_spmm2():
    @functools.partial(
        pl.kernel,
        mesh=_sc_mesh(),
        compiler_params=pltpu.CompilerParams(needs_layout_passes=False),
        out_type=jax.ShapeDtypeStruct((SC_CORES, B * L, H), f32),
        scratch_types=list(_SC_SCRATCH),
    )
    def spmm2(t_hbm, colrow_hbm, w_hbm, bidx_hbm, out_hbm,
              cf, rw, wf, rb0, rb1, rb2, acc,
              ps0, ps1, ps2, ps3, ps4, ps5,
              gs0, gs1, gs2, ss0, ss1, ss2):
        cid = lax.axis_index("c")
        sid = lax.axis_index("s")
        wid = cid * SC_SUB + sid

        _zero_acc(rb0, acc, sid)
        plsc.subcore_barrier()

        _spmm_accumulate(t_hbm, colrow_hbm, w_hbm, wid, cf, rw, wf,
                         [rb0, rb1, rb2], acc,
                         [ps0, ps1, ps2, ps3, ps4, ps5],
                         [gs0, gs1, gs2], [ss0, ss1, ss2])
        plsc.subcore_barrier()

        def gchunk(g, carry):
            pltpu.async_copy(bidx_hbm.at[pl.ds(sid * GB + g * K, K)],
                             cf.at[pl.ds(0, K)], ps0).wait()
            idx = cf.at[pl.ds(0, K)]
            pltpu.async_copy(acc.at[idx], rb0, gs0).wait()
            pltpu.sync_copy(rb0, out_hbm.at[cid, pl.ds(sid * GB + g * K, K)])
            return carry

        lax.fori_loop(0, GCH, gchunk, 0)

    return spmm2


_SPMM1 = _make_spmm1()
_SPMM2 = _make_spmm2()


def kernel(A1_tensor, edge_index, edge_weight, batch_idx, n, Lin1, Lin1_bias,
           gc1_weight, gc1_bias, gc2_weight, gc2_bias, weight2, bias2,
           liner1_weight, liner1_bias, classifier, classifier_bias):
    pad = EPT_P - EPT
    col3 = jnp.pad(edge_index[1].reshape(NW, EPT),
                   ((0, 0), (0, pad))).reshape(NW, NCH, 1, K)
    row3 = jnp.pad(edge_index[0].reshape(NW, EPT),
                   ((0, 0), (0, pad))).reshape(NW, NCH, 1, K)
    colrow = jnp.concatenate([col3, row3], axis=2)
    ew2 = jnp.pad(edge_weight.reshape(NW, EPT),
                  ((0, 0), (0, pad))).reshape(NW, NCH, K)

    wcat = jnp.concatenate([Lin1, gc1_weight], axis=1)
    bcat = jnp.concatenate([Lin1_bias, gc1_bias])[None, :]

    a1, t1 = _tc_in(A1_tensor, wcat, bcat, n)
    p = _SPMM1(t1, colrow, ew2)
    t2 = _tc_mid(p, a1, n, gc2_weight, gc2_bias[None, :])

    bidxT = batch_idx[:, 0, :].T.reshape(B * L)
    selp = _SPMM2(t2, colrow, ew2, bidxT)

    w1r = liner1_weight.T.reshape(L + 1, 64, H)
    z, am = _tc_head(selp.reshape(SC_CORES, L, B, H), weight2, bias2[None, :],
                     w1r, liner1_bias[None, :], classifier,
                     classifier_bias[None, :])
    return (am.reshape(B), z)


# speedup vs baseline: 1.0279x; 1.0279x over previous
import functools

import jax
import jax.numpy as jnp
from jax import lax
from jax.experimental import pallas as pl
from jax.experimental.pallas import tpu as pltpu
from jax.experimental.pallas import tpu_sc as plsc

N = 10000
E = 320000
D = 128
H = 128
L = 20
B = 1024
NCLS = 10

SC_CORES = 2
SC_SUB = 16
NW = SC_CORES * SC_SUB
K = 128
EPT = E // NW
NCH = 84
EPT_P = NCH * K
RPT = 624
RTAIL = N - RPT * SC_SUB
GB = B * L // SC_SUB
GCH = GB // K

f32 = jnp.float32


def _tc_in_body(x_ref, wcat_ref, bcat_ref, n_ref, a1_ref, t1_ref):
    y = jnp.dot(x_ref[...], wcat_ref[...], preferred_element_type=f32)
    y = y + bcat_ref[...]
    a1_ref[...] = y[:, :H] * n_ref[...]
    t1_ref[...] = y[:, H:]


def _tc_in(A1, wcat, bcat, n):
    blk = 1000
    return pl.pallas_call(
        _tc_in_body,
        grid=(N // blk,),
        in_specs=[
            pl.BlockSpec((blk, D), lambda i: (i, 0)),
            pl.BlockSpec((D, 2 * H), lambda i: (0, 0)),
            pl.BlockSpec((1, 2 * H), lambda i: (0, 0)),
            pl.BlockSpec((blk, 1), lambda i: (i, 0)),
        ],
        out_specs=[
            pl.BlockSpec((blk, H), lambda i: (i, 0)),
            pl.BlockSpec((blk, H), lambda i: (i, 0)),
        ],
        out_shape=[
            jax.ShapeDtypeStruct((N, H), f32),
            jax.ShapeDtypeStruct((N, H), f32),
        ],
    )(A1, wcat, bcat, n)


def _tc_mid_body(p0_ref, p1_ref, a1_ref, n_ref, w_ref, b_ref, t2_ref):
    nn = n_ref[...]
    x1 = a1_ref[...] + (p0_ref[0] + p1_ref[0]) * (1.0 - nn)
    t2_ref[...] = jnp.dot(x1, w_ref[...], preferred_element_type=f32) + b_ref[...]


def _tc_mid(p, a1, n, w, b):
    blk = 1000
    return pl.pallas_call(
        _tc_mid_body,
        grid=(N // blk,),
        in_specs=[
            pl.BlockSpec((1, blk, H), lambda i: (0, i, 0)),
            pl.BlockSpec((1, blk, H), lambda i: (1, i, 0)),
            pl.BlockSpec((blk, H), lambda i: (i, 0)),
            pl.BlockSpec((blk, 1), lambda i: (i, 0)),
            pl.BlockSpec((H, H), lambda i: (0, 0)),
            pl.BlockSpec((1, H), lambda i: (0, 0)),
        ],
        out_specs=pl.BlockSpec((blk, H), lambda i: (i, 0)),
        out_shape=jax.ShapeDtypeStruct((N, H), f32),
    )(p, p, a1, n, w, b)


def _tc_head_body(s_ref, w2_ref, b2_ref, w1r_ref, l1b_ref, cls_ref, clsb_ref,
                  z_ref, am_ref):
    w2 = w2_ref[...]
    b2 = b2_ref[...]
    z = jnp.broadcast_to(l1b_ref[...], (B, H))
    ssum = jnp.zeros((B, H), f32)
    for j in range(L):
        s = s_ref[0, j] + s_ref[1, j]
        ssum = ssum + s
        y = jnp.dot(s, w2, preferred_element_type=f32) + b2
        z = z + jnp.dot(y, w1r_ref[j], preferred_element_type=f32)
    y = jnp.dot(ssum, w2, preferred_element_type=f32) + b2
    z = z + jnp.dot(y, w1r_ref[L], preferred_element_type=f32)
    z_ref[...] = z
    logits = jnp.dot(z, cls_ref[...], preferred_element_type=f32) + clsb_ref[...]
    am_ref[...] = jnp.argmax(logits, axis=-1).astype(jnp.int32)[None, :]


def _tc_head(selp, w2, b2, w1r, l1b, cls, clsb):
    return pl.pallas_call(
        _tc_head_body,
        out_shape=[
            jax.ShapeDtypeStruct((B, H), f32),
            jax.ShapeDtypeStruct((1, B), jnp.int32),
        ],
    )(selp, w2, b2, w1r, l1b, cls, clsb)


def _sc_mesh():
    return plsc.VectorSubcoreMesh(core_axis_name="c", subcore_axis_name="s",
                                  num_cores=SC_CORES, num_subcores=SC_SUB)


def _zero_acc(zbuf, acc, sid, sem):
    def zrow(i, carry):
        for hh in range(H // 16):
            zbuf[i, pl.ds(hh * 16, 16)] = jnp.zeros((16,), f32)
        return carry

    lax.fori_loop(0, K, zrow, 0)
    r0 = sid * RPT
    nfull = RPT // K
    rem = RPT % K
    descs = []
    for kb in range(nfull):
        descs.append(pltpu.async_copy(
            zbuf, acc.at[pl.ds(r0 + kb * K, K)], sem))
    if rem:
        descs.append(pltpu.async_copy(
            zbuf.at[pl.ds(0, rem)], acc.at[pl.ds(r0 + nfull * K, rem)], sem))

    @pl.when(sid == SC_SUB - 1)
    def _():
        pltpu.async_copy(zbuf.at[pl.ds(0, RTAIL)],
                         acc.at[pl.ds(RPT * SC_SUB, RTAIL)], sem).wait()
    for d in descs:
        d.wait()


def _scale_chunk(rb, wf, c3):
    woff = c3 * K

    def grp(g, carry):
        e0 = g * 8
        for j in range(8):
            wv = plsc.load_gather(
                wf, [jnp.full((16,), woff + e0 + j, jnp.int32)])
            for hh in range(H // 16):
                sl = pl.ds(hh * 16, 16)
                rb[e0 + j, sl] = rb[e0 + j, sl] * wv
        return carry

    lax.fori_loop(0, K // 8, grp, 0)


def _spmm_accumulate(t_hbm, colrow_hbm, w_hbm, wid, cf, rw, wf, rbufs, acc,
                     gsems, ssem):
    def bundle(c, b3, cslot):
        idx = cf.at[pl.ds((c % 6) * K, K)]
        pltpu.async_copy(t_hbm.at[idx], rbufs[b3], gsems[b3])
        pltpu.async_copy(colrow_hbm.at[wid, c, 1], rw.at[b3], gsems[b3])
        pltpu.async_copy(w_hbm.at[wid, c], wf.at[pl.ds(b3 * K, K)], gsems[b3])
        pltpu.async_copy(colrow_hbm.at[wid, c + 2, 0],
                         cf.at[pl.ds(cslot * K, K)], gsems[b3])

    def wait_bundle(b3):
        pltpu.make_async_copy(t_hbm.at[pl.ds(0, K)], rbufs[b3],
                              gsems[b3]).wait()
        pltpu.make_async_copy(colrow_hbm.at[wid, 0, 1], rw.at[b3],
                              gsems[b3]).wait()
        pltpu.make_async_copy(w_hbm.at[wid, 0], wf.at[pl.ds(0, K)],
                              gsems[b3]).wait()
        pltpu.make_async_copy(colrow_hbm.at[wid, 0, 0], cf.at[pl.ds(0, K)],
                              gsems[b3]).wait()

    pltpu.sync_copy(colrow_hbm.at[wid, 0, 0], cf.at[pl.ds(0, K)])
    pltpu.sync_copy(colrow_hbm.at[wid, 1, 0], cf.at[pl.ds(K, K)])
    bundle(0, 0, 2)
    bundle(1, 1, 3)

    def step(k, carry):
        sdesc = None
        for bb in range(6):
            c = 6 * k + bb
            b3 = bb % 3
            b2 = (bb + 2) % 3
            wait_bundle(b3)
            _scale_chunk(rbufs[b3], wf, b3)
            if sdesc is not None:
                sdesc.wait()

            @pl.when(c + 2 < NCH)
            def _():
                bundle(c + 2, b2, (bb + 4) % 6)

            if bb < 5:
                sdesc = pltpu.async_copy(rbufs[b3], acc.at[rw.at[b3]], ssem,
                                         add=True)
            else:
                pltpu.sync_copy(rbufs[b3], acc.at[rw.at[b3]], add=True)
                sdesc = None
        return carry

    lax.fori_loop(0, NCH // 6, step, 0)


def _acc_writeout(acc, out_hbm, cid, sid, sem):
    r0 = sid * RPT
    nfull = RPT // K
    rem = RPT % K
    descs = []
    for kb in range(nfull):
        sl = pl.ds(r0 + kb * K, K)
        descs.append(pltpu.async_copy(acc.at[sl], out_hbm.at[cid, sl], sem))
    if rem:
        sl = pl.ds(r0 + nfull * K, rem)
        descs.append(pltpu.async_copy(acc.at[sl], out_hbm.at[cid, sl], sem))

    @pl.when(sid == SC_SUB - 1)
    def _():
        sl = pl.ds(RPT * SC_SUB, RTAIL)
        pltpu.async_copy(acc.at[sl], out_hbm.at[cid, sl], sem).wait()
    for d in descs:
        d.wait()


_SC_SCRATCH = [
    pltpu.VMEM((6 * K,), jnp.int32),
    pltpu.VMEM((3, K), jnp.int32),
    pltpu.VMEM((3 * K,), f32),
    pltpu.VMEM((K, H), f32),
    pltpu.VMEM((K, H), f32),
    pltpu.VMEM((K, H), f32),
    pltpu.VMEM_SHARED((N, H), f32),
    pltpu.SemaphoreType.DMA,
    pltpu.SemaphoreType.DMA,
    pltpu.SemaphoreType.DMA,
    pltpu.SemaphoreType.DMA,
]


def _make_spmm1():
    @functools.partial(
        pl.kernel,
        mesh=_sc_mesh(),
        compiler_params=pltpu.CompilerParams(needs_layout_passes=False),
        out_type=jax.ShapeDtypeStruct((SC_CORES, N, H), f32),
        scratch_types=list(_SC_SCRATCH),
    )
    def spmm1(t_hbm, colrow_hbm, w_hbm, out_hbm,
              cf, rw, wf, rb0, rb1, rb2, acc,
              gs0, gs1, gs2, ss0):
        cid = lax.axis_index("c")
        sid = lax.axis_index("s")
        wid = cid * SC_SUB + sid

        _zero_acc(rb0, acc, sid, gs0)
        plsc.subcore_barrier()

        _spmm_accumulate(t_hbm, colrow_hbm, w_hbm, wid, cf, rw, wf,
                         [rb0, rb1, rb2], acc, [gs0, gs1, gs2], ss0)
        plsc.subcore_barrier()
        _acc_writeout(acc, out_hbm, cid, sid, gs0)

    return spmm1


def _make_spmm2():
    @functools.partial(
        pl.kernel,
        mesh=_sc_mesh(),
        compiler_params=pltpu.CompilerParams(needs_layout_passes=False),
        out_type=jax.ShapeDtypeStruct((SC_CORES, B * L, H), f32),
        scratch_types=list(_SC_SCRATCH),
    )
    def spmm2(t_hbm, colrow_hbm, w_hbm, bidx_hbm, out_hbm,
              cf, rw, wf, rb0, rb1, rb2, acc,
              gs0, gs1, gs2, ss0):
        cid = lax.axis_index("c")
        sid = lax.axis_index("s")
        wid = cid * SC_SUB + sid

        _zero_acc(rb0, acc, sid, gs0)
        plsc.subcore_barrier()

        _spmm_accumulate(t_hbm, colrow_hbm, w_hbm, wid, cf, rw, wf,
                         [rb0, rb1, rb2], acc, [gs0, gs1, gs2], ss0)
        plsc.subcore_barrier()

        rbufs = [rb0, rb1]
        idescs = [None, None]
        gdescs = [None, None]
        odescs = [None, None]
        for g in range(GCH):
            sg = g % 2
            if g < 2:
                idescs[sg] = pltpu.async_copy(
                    bidx_hbm.at[pl.ds(sid * GB + g * K, K)],
                    cf.at[pl.ds(sg * K, K)], gsems_ := [gs0, gs1][sg])
        for g in range(GCH):
            sg = g % 2
            gsem = [gs0, gs1][sg]
            idescs[sg].wait()
            if odescs[sg] is not None:
                odescs[sg].wait()
            gdescs[sg] = pltpu.async_copy(
                acc.at[cf.at[pl.ds(sg * K, K)]], rbufs[sg], gsem)
            gdescs[sg].wait()
            if g + 2 < GCH:
                idescs[sg] = pltpu.async_copy(
                    bidx_hbm.at[pl.ds(sid * GB + (g + 2) * K, K)],
                    cf.at[pl.ds(sg * K, K)], gsem)
            odescs[sg] = pltpu.async_copy(
                rbufs[sg], out_hbm.at[cid, pl.ds(sid * GB + g * K, K)], ss0)
        for sg in range(2):
            if odescs[sg] is not None:
                odescs[sg].wait()

    return spmm2


_SPMM1 = _make_spmm1()
_SPMM2 = _make_spmm2()


def kernel(A1_tensor, edge_index, edge_weight, batch_idx, n, Lin1, Lin1_bias,
           gc1_weight, gc1_bias, gc2_weight, gc2_bias, weight2, bias2,
           liner1_weight, liner1_bias, classifier, classifier_bias):
    pad = EPT_P - EPT
    col3 = jnp.pad(edge_index[1].reshape(NW, EPT),
                   ((0, 0), (0, pad))).reshape(NW, NCH, 1, K)
    row3 = jnp.pad(edge_index[0].reshape(NW, EPT),
                   ((0, 0), (0, pad))).reshape(NW, NCH, 1, K)
    colrow = jnp.concatenate([col3, row3], axis=2)
    colrow = jnp.pad(colrow, ((0, 0), (0, 2), (0, 0), (0, 0)))
    ew2 = jnp.pad(edge_weight.reshape(NW, EPT),
                  ((0, 0), (0, pad))).reshape(NW, NCH, K)

    wcat = jnp.concatenate([Lin1, gc1_weight], axis=1)
    bcat = jnp.concatenate([Lin1_bias, gc1_bias])[None, :]

    a1, t1 = _tc_in(A1_tensor, wcat, bcat, n)
    p = _SPMM1(t1, colrow, ew2)
    t2 = _tc_mid(p, a1, n, gc2_weight, gc2_bias[None, :])

    bidxT = batch_idx[:, 0, :].T.reshape(B * L)
    selp = _SPMM2(t2, colrow, ew2, bidxT)

    w1r = liner1_weight.T.reshape(L + 1, 64, H)
    z, am = _tc_head(selp.reshape(SC_CORES, L, B, H), weight2, bias2[None, :],
                     w1r, liner1_bias[None, :], classifier,
                     classifier_bias[None, :])
    return (am.reshape(B), z)


# speedup vs baseline: 2.2645x; 2.2030x over previous
import functools

import jax
import jax.numpy as jnp
from jax import lax
from jax.experimental import pallas as pl
from jax.experimental.pallas import tpu as pltpu
from jax.experimental.pallas import tpu_sc as plsc

N = 10000
E = 320000
D = 128
H = 128
L = 20
B = 1024
NCLS = 10

SC_CORES = 2
SC_SUB = 16
NW = SC_CORES * SC_SUB
K = 128
EPT = E // NW
NCH = 80
NPH = NCH // 2
EPT_P = NCH * K
EPH = NPH * K
RPT = 624
RTAIL = N - RPT * SC_SUB
GB = B * L // SC_SUB
GCH = GB // K

f32 = jnp.float32


def _tc_in_body(x_ref, wcat_ref, bcat_ref, n_ref, a1_ref, t1_ref):
    y = jnp.dot(x_ref[...], wcat_ref[...], preferred_element_type=f32)
    y = y + bcat_ref[...]
    a1_ref[...] = y[:, :H] * n_ref[...]
    t1_ref[...] = y[:, H:]


def _tc_in(A1, wcat, bcat, n):
    blk = 1000
    return pl.pallas_call(
        _tc_in_body,
        grid=(N // blk,),
        in_specs=[
            pl.BlockSpec((blk, D), lambda i: (i, 0)),
            pl.BlockSpec((D, 2 * H), lambda i: (0, 0)),
            pl.BlockSpec((1, 2 * H), lambda i: (0, 0)),
            pl.BlockSpec((blk, 1), lambda i: (i, 0)),
        ],
        out_specs=[
            pl.BlockSpec((blk, H), lambda i: (i, 0)),
            pl.BlockSpec((blk, H), lambda i: (i, 0)),
        ],
        out_shape=[
            jax.ShapeDtypeStruct((N, H), f32),
            jax.ShapeDtypeStruct((N, H), f32),
        ],
    )(A1, wcat, bcat, n)


def _tc_mid_body(p0_ref, p1_ref, a1_ref, n_ref, w_ref, b_ref, t2_ref):
    nn = n_ref[...]
    x1 = a1_ref[...] + (p0_ref[0] + p1_ref[0]) * (1.0 - nn)
    t2_ref[...] = jnp.dot(x1, w_ref[...], preferred_element_type=f32) + b_ref[...]


def _tc_mid(p, a1, n, w, b):
    blk = 1000
    return pl.pallas_call(
        _tc_mid_body,
        grid=(N // blk,),
        in_specs=[
            pl.BlockSpec((1, blk, H), lambda i: (0, i, 0)),
            pl.BlockSpec((1, blk, H), lambda i: (1, i, 0)),
            pl.BlockSpec((blk, H), lambda i: (i, 0)),
            pl.BlockSpec((blk, 1), lambda i: (i, 0)),
            pl.BlockSpec((H, H), lambda i: (0, 0)),
            pl.BlockSpec((1, H), lambda i: (0, 0)),
        ],
        out_specs=pl.BlockSpec((blk, H), lambda i: (i, 0)),
        out_shape=jax.ShapeDtypeStruct((N, H), f32),
    )(p, p, a1, n, w, b)


def _tc_head_body(s_ref, w2_ref, b2_ref, w1r_ref, l1b_ref, cls_ref, clsb_ref,
                  z_ref, am_ref):
    w2 = w2_ref[...]
    b2 = b2_ref[...]
    z = jnp.broadcast_to(l1b_ref[...], (B, H))
    ssum = jnp.zeros((B, H), f32)
    for j in range(L):
        s = s_ref[0, j] + s_ref[1, j]
        ssum = ssum + s
        y = jnp.dot(s, w2, preferred_element_type=f32) + b2
        z = z + jnp.dot(y, w1r_ref[j], preferred_element_type=f32)
    y = jnp.dot(ssum, w2, preferred_element_type=f32) + b2
    z = z + jnp.dot(y, w1r_ref[L], preferred_element_type=f32)
    z_ref[...] = z
    logits = jnp.dot(z, cls_ref[...], preferred_element_type=f32) + clsb_ref[...]
    am_ref[...] = jnp.argmax(logits, axis=-1).astype(jnp.int32)[None, :]


def _tc_head(selp, w2, b2, w1r, l1b, cls, clsb):
    return pl.pallas_call(
        _tc_head_body,
        out_shape=[
            jax.ShapeDtypeStruct((B, H), f32),
            jax.ShapeDtypeStruct((1, B), jnp.int32),
        ],
    )(selp, w2, b2, w1r, l1b, cls, clsb)


def _sc_mesh():
    return plsc.VectorSubcoreMesh(core_axis_name="c", subcore_axis_name="s",
                                  num_cores=SC_CORES, num_subcores=SC_SUB)


def _zero_acc(zbuf, acc, sid, sem):
    def zrow(i, carry):
        for hh in range(H // 16):
            zbuf[i, pl.ds(hh * 16, 16)] = jnp.zeros((16,), f32)
        return carry

    lax.fori_loop(0, K, zrow, 0)
    r0 = sid * RPT
    full = RPT // K
    rem = RPT % K
    descs = []
    for kb in range(full):
        descs.append(pltpu.async_copy(zbuf, acc.at[pl.ds(r0 + kb * K, K)], sem))
    if rem:
        descs.append(pltpu.async_copy(zbuf.at[pl.ds(0, rem)],
                                      acc.at[pl.ds(r0 + full * K, rem)], sem))

    @pl.when(sid == SC_SUB - 1)
    def _():
        pltpu.async_copy(zbuf.at[pl.ds(0, RTAIL)],
                         acc.at[pl.ds(RPT * SC_SUB, RTAIL)], sem).wait()
    for d in descs:
        d.wait()


def _scale_chunk(rb, w_v, cph):
    def scale(e, carry):
        wv = plsc.load_gather(w_v, [jnp.full((16,), cph * K + e, jnp.int32)])
        for hh in range(H // 16):
            sl = pl.ds(hh * 16, 16)
            rb[e, sl] = rb[e, sl] * wv
        return carry

    lax.fori_loop(0, K, scale, 0, unroll=2)


def _spmm_accumulate(t_hbm, cols_hbm, rows_hbm, w_hbm, wid,
                     col_v, row_v, w_v, rbufs, acc, gsems):
    def gather(cph, s2):
        idx = col_v.at[pl.ds(cph * K, K)]
        pltpu.async_copy(t_hbm.at[idx], rbufs[s2], gsems[s2])

    def wait_gather(s2):
        pltpu.make_async_copy(t_hbm.at[pl.ds(0, K)], rbufs[s2],
                              gsems[s2]).wait()

    for ph in range(2):
        pltpu.sync_copy(cols_hbm.at[wid, ph], col_v)
        pltpu.sync_copy(rows_hbm.at[wid, ph], row_v)
        pltpu.sync_copy(w_hbm.at[wid, ph], w_v)
        gather(0, 0)
        gather(1, 1)

        def step(k, carry):
            for ss in range(2):
                cph = 2 * k + ss
                wait_gather(ss)
                _scale_chunk(rbufs[ss], w_v, cph)
                pltpu.sync_copy(rbufs[ss], acc.at[row_v.at[cph]], add=True)

                @pl.when(cph + 2 < NPH)
                def _():
                    gather(cph + 2, ss)
            return carry

        lax.fori_loop(0, NPH // 2, step, 0)


def _acc_writeout(acc, out_hbm, cid, sid, sem):
    r0 = sid * RPT
    full = RPT // K
    rem = RPT % K
    descs = []
    for kb in range(full):
        sl = pl.ds(r0 + kb * K, K)
        descs.append(pltpu.async_copy(acc.at[sl], out_hbm.at[cid, sl], sem))
    if rem:
        sl = pl.ds(r0 + full * K, rem)
        descs.append(pltpu.async_copy(acc.at[sl], out_hbm.at[cid, sl], sem))

    @pl.when(sid == SC_SUB - 1)
    def _():
        sl = pl.ds(RPT * SC_SUB, RTAIL)
        pltpu.async_copy(acc.at[sl], out_hbm.at[cid, sl], sem).wait()
    for d in descs:
        d.wait()


_SC_SCRATCH = [
    pltpu.VMEM((EPH,), jnp.int32),
    pltpu.VMEM((NPH, K), jnp.int32),
    pltpu.VMEM((EPH,), f32),
    pltpu.VMEM((K, H), f32),
    pltpu.VMEM((K, H), f32),
    pltpu.VMEM_SHARED((N, H), f32),
    pltpu.SemaphoreType.DMA,
    pltpu.SemaphoreType.DMA,
]


def _make_spmm1():
    @functools.partial(
        pl.kernel,
        mesh=_sc_mesh(),
        compiler_params=pltpu.CompilerParams(needs_layout_passes=False),
        out_type=jax.ShapeDtypeStruct((SC_CORES, N, H), f32),
        scratch_types=list(_SC_SCRATCH),
    )
    def spmm1(t_hbm, cols_hbm, rows_hbm, w_hbm, out_hbm,
              col_v, row_v, w_v, rb0, rb1, acc, gs0, gs1):
        cid = lax.axis_index("c")
        sid = lax.axis_index("s")
        wid = cid * SC_SUB + sid

        _zero_acc(rb0, acc, sid, gs0)
        plsc.subcore_barrier()

        _spmm_accumulate(t_hbm, cols_hbm, rows_hbm, w_hbm, wid,
                         col_v, row_v, w_v, [rb0, rb1], acc, [gs0, gs1])
        plsc.subcore_barrier()
        _acc_writeout(acc, out_hbm, cid, sid, gs0)

    return spmm1


def _make_spmm2():
    @functools.partial(
        pl.kernel,
        mesh=_sc_mesh(),
        compiler_params=pltpu.CompilerParams(needs_layout_passes=False),
        out_type=jax.ShapeDtypeStruct((SC_CORES, B * L, H), f32),
        scratch_types=list(_SC_SCRATCH) + [pltpu.VMEM((GB,), jnp.int32)],
    )
    def spmm2(t_hbm, cols_hbm, rows_hbm, w_hbm, bidx_hbm, out_hbm,
              col_v, row_v, w_v, rb0, rb1, acc, gs0, gs1, bidx_v):
        cid = lax.axis_index("c")
        sid = lax.axis_index("s")
        wid = cid * SC_SUB + sid

        _zero_acc(rb0, acc, sid, gs0)
        pltpu.sync_copy(bidx_hbm.at[pl.ds(sid * GB, GB)], bidx_v)
        plsc.subcore_barrier()

        _spmm_accumulate(t_hbm, cols_hbm, rows_hbm, w_hbm, wid,
                         col_v, row_v, w_v, [rb0, rb1], acc, [gs0, gs1])
        plsc.subcore_barrier()

        rbufs = [rb0, rb1]
        gsems = [gs0, gs1]
        gd = [
            pltpu.async_copy(acc.at[bidx_v.at[pl.ds(0, K)]], rb0, gs0),
            pltpu.async_copy(acc.at[bidx_v.at[pl.ds(K, K)]], rb1, gs1),
        ]
        od = [None, None]
        for g in range(GCH):
            sg = g % 2
            gd[sg].wait()
            od[sg] = pltpu.async_copy(
                rbufs[sg], out_hbm.at[cid, pl.ds(sid * GB + g * K, K)],
                gsems[sg])
            if g + 2 < GCH:
                od[sg].wait()
                gd[sg] = pltpu.async_copy(
                    acc.at[bidx_v.at[pl.ds((g + 2) * K, K)]], rbufs[sg],
                    gsems[sg])
        od[0].wait()
        od[1].wait()

    return spmm2


_SPMM1 = _make_spmm1()
_SPMM2 = _make_spmm2()


def kernel(A1_tensor, edge_index, edge_weight, batch_idx, n, Lin1, Lin1_bias,
           gc1_weight, gc1_bias, gc2_weight, gc2_bias, weight2, bias2,
           liner1_weight, liner1_bias, classifier, classifier_bias):
    pad = EPT_P - EPT
    col3 = jnp.pad(edge_index[1].reshape(NW, EPT),
                   ((0, 0), (0, pad))).reshape(NW, 2, EPH)
    row4 = jnp.pad(edge_index[0].reshape(NW, EPT),
                   ((0, 0), (0, pad))).reshape(NW, 2, NPH, K)
    ew3 = jnp.pad(edge_weight.reshape(NW, EPT),
                  ((0, 0), (0, pad))).reshape(NW, 2, EPH)

    wcat = jnp.concatenate([Lin1, gc1_weight], axis=1)
    bcat = jnp.concatenate([Lin1_bias, gc1_bias])[None, :]

    a1, t1 = _tc_in(A1_tensor, wcat, bcat, n)
    p = _SPMM1(t1, col3, row4, ew3)
    t2 = _tc_mid(p, a1, n, gc2_weight, gc2_bias[None, :])

    bidxT = batch_idx[:, 0, :].T.reshape(B * L)
    selp = _SPMM2(t2, col3, row4, ew3, bidxT)

    w1r = liner1_weight.T.reshape(L + 1, 64, H)
    z, am = _tc_head(selp.reshape(SC_CORES, L, B, H), weight2, bias2[None, :],
                     w1r, liner1_bias[None, :], classifier,
                     classifier_bias[None, :])
    return (am.reshape(B), z)
